# baseline (speedup 1.0000x reference)
import functools

import jax
import jax.numpy as jnp
from jax import lax
from jax.experimental import pallas as pl
from jax.experimental.pallas import tpu as pltpu
from jax.experimental.pallas import tpu_sc as plsc

_L = 16
_NBUF = 3


def _make_sc_lookup_add(N, D, rows_per_w, C, nc, BB, SS):
    NCH = -(-rows_per_w // C)
    last_off = rows_per_w - C
    vregs_per_row = D // _L
    mesh = plsc.VectorSubcoreMesh(core_axis_name="c", subcore_axis_name="s")

    @functools.partial(
        pl.kernel,
        out_type=jax.ShapeDtypeStruct((N, D), jnp.float32),
        mesh=mesh,
        scratch_types=(
            [pltpu.VMEM((rows_per_w,), jnp.int32)]
            + [pltpu.VMEM((rows_per_w,), jnp.int32)]
            + [pltpu.VMEM((C, D), jnp.float32) for _ in range(2 * _NBUF)]
            + [pltpu.SemaphoreType.DMA for _ in range(3 * _NBUF)]
        ),
    )
    def body(x_hbm, pos_hbm, emb_hbm, out_hbm, addr_v, idx_v, *bufs):
        XV = bufs[0:3]
        RV = bufs[3:6]
        SX = bufs[6:9]
        SG = bufs[9:12]
        SO = bufs[12:15]

        wid = lax.axis_index("s") * nc + lax.axis_index("c")
        w_base = wid * rows_per_w

        lane = lax.iota(jnp.int32, _L)

        def addr_blk(k, carry):
            r = w_base + k * _L + lane
            b = lax.rem(r, jnp.int32(BB))
            s = lax.div(r, jnp.int32(BB))
            addr_v[pl.ds(k * _L, _L)] = b * SS + s
            return carry

        lax.fori_loop(0, rows_per_w // _L, addr_blk, 0)
        pltpu.async_copy(pos_hbm.at[addr_v], idx_v, SX[0]).wait()

        def coff(c):
            if isinstance(c, int):
                return min(c * C, last_off)
            return pl.multiple_of(jnp.minimum(c * C, last_off), 8)

        def start_in(c, b):
            off = coff(c)
            pltpu.async_copy(x_hbm.at[pl.ds(w_base + off, C)], XV[b], SX[b])
            pltpu.async_copy(emb_hbm.at[idx_v.at[pl.ds(off, C)]], RV[b], SG[b])

        def wait_in(b):
            pltpu.make_async_copy(x_hbm.at[pl.ds(0, C)], XV[b], SX[b]).wait()
            pltpu.make_async_copy(emb_hbm.at[idx_v.at[pl.ds(0, C)]], RV[b],
                                  SG[b]).wait()

        def compute(b):
            def row(r, carry):
                @plsc.parallel_loop(0, vregs_per_row, step=1, unroll=8)
                def _vloop(j):
                    sl = pl.ds(j * _L, _L)
                    plsc.addupdate(XV[b].at[r, sl], RV[b][r, sl])

                return carry

            lax.fori_loop(0, C, row, 0)

        def start_out(c, b):
            pltpu.async_copy(XV[b], out_hbm.at[pl.ds(w_base + coff(c), C)],
                             SO[b])

        def wait_out(b):
            pltpu.make_async_copy(XV[b], out_hbm.at[pl.ds(0, C)], SO[b]).wait()

        def step(c, b, first=False, last=False):
            wait_in(b)
            compute(b)
            start_out(c, b)
            b2 = (b + 2) % _NBUF
            if not last:
                if not first:
                    wait_out(b2)
                start_in(c + 2, b2)

        G = (NCH - 5) // 3
        start_in(0, 0)
        start_in(1, 1)
        step(0, 0, first=True)
        step(1, 1)
        step(2, 2)

        def g_body(g, carry):
            c0 = 3 * g
            step(c0, 0)
            step(c0 + 1, 1)
            step(c0 + 2, 2)
            return carry

        lax.fori_loop(1, 1 + G, g_body, 0)
        for c in range(3 + 3 * G, NCH):
            step(c, c % 3, last=(c + 2 >= NCH))
        for b in range(_NBUF):
            wait_out(b)

    return body


def kernel(x, pos, emb):
    S, B, D = x.shape
    N = S * B

    info = plsc.get_sparse_core_info()
    nc, ns = info.num_cores, info.num_subcores
    rows_per_w = N // (nc * ns)
    chunk_rows = 16

    pos_flat = pos.reshape(B * S)
    x2 = x.reshape(N, D)

    fn = _make_sc_lookup_add(N, D, rows_per_w, chunk_rows, nc, B, S)
    out = fn(x2, pos_flat, emb)
    return out.reshape(S, B, D)


# speedup vs baseline: 1.0891x; 1.0891x over previous
import functools

import jax
import jax.numpy as jnp
from jax import lax
from jax.experimental import pallas as pl
from jax.experimental.pallas import tpu as pltpu
from jax.experimental.pallas import tpu_sc as plsc

_L = 16
_NBUF = 3


def _make_sc_lookup_add(N, D, rows_per_w, C, nc, BB, SS):
    NCH = -(-rows_per_w // C)
    last_off = rows_per_w - C
    vregs_per_row = D // _L
    mesh = plsc.VectorSubcoreMesh(core_axis_name="c", subcore_axis_name="s")

    @functools.partial(
        pl.kernel,
        out_type=jax.ShapeDtypeStruct((N, D), jnp.float32),
        mesh=mesh,
        scratch_types=(
            [pltpu.VMEM((rows_per_w,), jnp.int32)]
            + [pltpu.VMEM((C, D), jnp.float32) for _ in range(2 * _NBUF)]
            + [pltpu.SemaphoreType.DMA for _ in range(3 * _NBUF)]
        ),
    )
    def body(x_hbm, idx_hbm, emb_hbm, out_hbm, idx_v, *bufs):
        XV = bufs[0:3]
        RV = bufs[3:6]
        SX = bufs[6:9]
        SG = bufs[9:12]
        SO = bufs[12:15]

        wid = lax.axis_index("s") * nc + lax.axis_index("c")
        w_base = wid * rows_per_w

        def coff(c):
            if isinstance(c, int):
                return min(c * C, last_off)
            return pl.multiple_of(jnp.minimum(c * C, last_off), 8)

        def start_in_x(c, b):
            pltpu.async_copy(x_hbm.at[pl.ds(w_base + coff(c), C)], XV[b], SX[b])

        def start_in_g(c, b):
            pltpu.async_copy(emb_hbm.at[idx_v.at[pl.ds(coff(c), C)]], RV[b],
                             SG[b])

        def start_in(c, b):
            start_in_x(c, b)
            start_in_g(c, b)

        def wait_in(b):
            pltpu.make_async_copy(x_hbm.at[pl.ds(0, C)], XV[b], SX[b]).wait()
            pltpu.make_async_copy(emb_hbm.at[idx_v.at[pl.ds(0, C)]], RV[b],
                                  SG[b]).wait()

        def compute(b):
            def row(r, carry):
                @plsc.parallel_loop(0, vregs_per_row, step=1, unroll=8)
                def _vloop(j):
                    sl = pl.ds(j * _L, _L)
                    plsc.addupdate(XV[b].at[r, sl], RV[b][r, sl])

                return carry

            lax.fori_loop(0, C, row, 0)

        def start_out(c, b):
            pltpu.async_copy(XV[b], out_hbm.at[pl.ds(w_base + coff(c), C)],
                             SO[b])

        def wait_out(b):
            pltpu.make_async_copy(XV[b], out_hbm.at[pl.ds(0, C)], SO[b]).wait()

        def step(c, b, first=False, last=False):
            wait_in(b)
            compute(b)
            start_out(c, b)
            b2 = (b + 2) % _NBUF
            if not last:
                if not first:
                    wait_out(b2)
                start_in(c + 2, b2)

        G = (NCH - 5) // 3
        for b in range(_NBUF):
            start_in_x(b, b)
        pltpu.sync_copy(idx_hbm.at[pl.ds(pl.multiple_of(w_base, 8), rows_per_w)],
                        idx_v)
        for b in range(_NBUF):
            start_in_g(b, b)
        step(0, 0, first=True, last=True)
        step(1, 1)
        step(2, 2)

        def g_body(g, carry):
            c0 = 3 * g
            step(c0, 0)
            step(c0 + 1, 1)
            step(c0 + 2, 2)
            return carry

        lax.fori_loop(1, 1 + G, g_body, 0)
        for c in range(3 + 3 * G, NCH):
            step(c, c % 3, last=(c + 2 >= NCH))
        for b in range(_NBUF):
            wait_out(b)

    return body


def kernel(x, pos, emb):
    S, B, D = x.shape
    N = S * B

    info = plsc.get_sparse_core_info()
    nc, ns = info.num_cores, info.num_subcores
    rows_per_w = N // (nc * ns)
    chunk_rows = 16

    idx = pos.T.reshape(N)
    x2 = x.reshape(N, D)

    fn = _make_sc_lookup_add(N, D, rows_per_w, chunk_rows, nc, B, S)
    out = fn(x2, idx, emb)
    return out.reshape(S, B, D)


# speedup vs baseline: 1.0892x; 1.0001x over previous
import functools

import jax
import jax.numpy as jnp
from jax import lax
from jax.experimental import pallas as pl
from jax.experimental.pallas import tpu as pltpu
from jax.experimental.pallas import tpu_sc as plsc

_L = 16
_NBUF = 3


def _make_sc_lookup_add(N, D, rows_per_w, C, nc):
    NCH = -(-rows_per_w // C)
    last_off = rows_per_w - C
    vregs_per_row = D // _L
    mesh = plsc.VectorSubcoreMesh(core_axis_name="c", subcore_axis_name="s")

    @functools.partial(
        pl.kernel,
        out_type=jax.ShapeDtypeStruct((N, D), jnp.float32),
        mesh=mesh,
        scratch_types=(
            [pltpu.VMEM((rows_per_w,), jnp.int32)]
            + [pltpu.VMEM((C, D), jnp.float32) for _ in range(2 * _NBUF)]
            + [pltpu.SemaphoreType.DMA for _ in range(3 * _NBUF)]
        ),
    )
    def body(x_hbm, idx_hbm, emb_hbm, out_hbm, idx_v, *bufs):
        XV = bufs[0:3]
        RV = bufs[3:6]
        SX = bufs[6:9]
        SG = bufs[9:12]
        SO = bufs[12:15]

        wid = lax.axis_index("s") * nc + lax.axis_index("c")
        w_base = wid * rows_per_w

        def coff(c):
            if isinstance(c, int):
                return min(c * C, last_off)
            return pl.multiple_of(jnp.minimum(c * C, last_off), 8)

        def start_in_x(c, b):
            pltpu.async_copy(x_hbm.at[pl.ds(w_base + coff(c), C)], XV[b], SX[b])

        def start_in_g(c, b):
            pltpu.async_copy(emb_hbm.at[idx_v.at[pl.ds(coff(c), C)]], RV[b],
                             SG[b])

        def start_in(c, b):
            start_in_x(c, b)
            start_in_g(c, b)

        def wait_in(b):
            pltpu.make_async_copy(x_hbm.at[pl.ds(0, C)], XV[b], SX[b]).wait()
            pltpu.make_async_copy(emb_hbm.at[idx_v.at[pl.ds(0, C)]], RV[b],
                                  SG[b]).wait()

        def compute(b):
            def row(r, carry):
                @plsc.parallel_loop(0, vregs_per_row, step=1, unroll=8)
                def _vloop(j):
                    sl = pl.ds(j * _L, _L)
                    plsc.addupdate(XV[b].at[r, sl], RV[b][r, sl])

                return carry

            lax.fori_loop(0, C, row, 0)

        def start_out(c, b):
            pltpu.async_copy(XV[b], out_hbm.at[pl.ds(w_base + coff(c), C)],
                             SO[b])

        def wait_out(b):
            pltpu.make_async_copy(XV[b], out_hbm.at[pl.ds(0, C)], SO[b]).wait()

        def step(c, b, first=False, last=False):
            wait_in(b)
            compute(b)
            start_out(c, b)
            b2 = (b + 2) % _NBUF
            if not last:
                if not first:
                    wait_out(b2)
                start_in(c + 2, b2)

        G = (NCH - 5) // 3
        for b in range(_NBUF):
            start_in_x(b, b)
        pltpu.sync_copy(idx_hbm.at[pl.ds(pl.multiple_of(w_base, 8), rows_per_w)],
                        idx_v)
        for b in range(_NBUF):
            start_in_g(b, b)
        step(0, 0, first=True, last=True)
        step(1, 1)
        step(2, 2)

        def g_body(g, carry):
            c0 = 3 * g
            step(c0, 0)
            step(c0 + 1, 1)
            step(c0 + 2, 2)
            return carry

        lax.fori_loop(1, 1 + G, g_body, 0)
        for c in range(3 + 3 * G, NCH):
            step(c, c % 3, last=(c + 2 >= NCH))
        for b in range(_NBUF):
            wait_out(b)

    return body


def kernel(x, pos, emb):
    S, B, D = x.shape
    N = S * B

    info = plsc.get_sparse_core_info()
    nc, ns = info.num_cores, info.num_subcores
    rows_per_w = N // (nc * ns)
    chunk_rows = 16

    idx = pos.T.reshape(N)
    x2 = x.reshape(N, D)

    fn = _make_sc_lookup_add(N, D, rows_per_w, chunk_rows, nc)
    out = fn(x2, idx, emb)
    return out.reshape(S, B, D)
